# baseline (speedup 1.0000x reference)
import jax
import jax.numpy as jnp
from jax import lax
from jax.experimental import pallas as pl
from jax.experimental.pallas import tpu as pltpu
from jax.experimental.pallas import tpu_sc as plsc

NUM_EDGES = 320000
NUM_RELS = 500
HIDDEN = 128
HW = HIDDEN // 2
NCH = HW // 16
NC = 2
NS = 16
NW = NC * NS
PER_W = NUM_EDGES // NW
BLK = 80
NBLK = PER_W // BLK


def _body(src_hbm, dst_hbm, typ_hbm, z_hbm, rel_hbm, out_hbm,
          sidx, didx, tidx,
          srows0, drows0, rrows0, srows1, drows1, rrows1,
          obuf, sem0, sem1):
    wid = lax.axis_index("s") * NC + lax.axis_index("c")
    wbase = wid * PER_W

    bufs = ((srows0, drows0, rrows0, sem0), (srows1, drows1, rrows1, sem1))
    lanes = lax.iota(jnp.int32, 16)
    last_lane = lanes == 15

    pltpu.sync_copy(src_hbm.at[pl.ds(wbase, PER_W)], sidx)
    pltpu.sync_copy(dst_hbm.at[pl.ds(wbase, PER_W)], didx)
    pltpu.sync_copy(typ_hbm.at[pl.ds(wbase, PER_W)], tidx)

    def issue(b, parity):
        sb, db, rb, sem = bufs[parity]
        sl = pl.ds(b * BLK, BLK)
        pltpu.async_copy(z_hbm.at[sidx.at[sl]], sb, sem)
        pltpu.async_copy(z_hbm.at[didx.at[sl]], db, sem)
        pltpu.async_copy(rel_hbm.at[tidx.at[sl]], rb, sem)

    def drain(b, parity):
        sb, db, rb, sem = bufs[parity]
        sl = pl.ds(b * BLK, BLK)
        pltpu.make_async_copy(z_hbm.at[sidx.at[sl]], sb, sem).wait()
        pltpu.make_async_copy(z_hbm.at[didx.at[sl]], db, sem).wait()
        pltpu.make_async_copy(rel_hbm.at[tidx.at[sl]], rb, sem).wait()

    def compute_blk(b, parity):
        sb, db, rb, _ = bufs[parity]
        obase = b * BLK

        def group(g, _):
            rows = g * 16 + lanes

            def wstep(w, accs):
                acc0, acc1 = accs
                cols = jnp.full((16,), w, jnp.int32)
                s = plsc.bitcast(plsc.load_gather(sb, [rows, cols]),
                                 jnp.bfloat16)
                d = plsc.bitcast(plsc.load_gather(db, [rows, cols]),
                                 jnp.bfloat16)
                r = plsc.bitcast(plsc.load_gather(rb, [rows, cols]),
                                 jnp.bfloat16)
                p = s * d * r
                p0, p1 = plsc.unpack(p, format=plsc.PackFormat.INTERLEAVED)
                return acc0 + p0, acc1 + p1

            z16 = jnp.zeros((16,), jnp.float32)
            acc0, acc1 = lax.fori_loop(0, HW, wstep, (z16, z16), unroll=2)
            obuf[pl.ds(obase + g * 16, 16)] = acc0 + acc1
            return 0

        lax.fori_loop(0, BLK // 16, group, 0)

    issue(0, 0)

    def pair(i, _):
        b0 = 2 * i
        b1 = 2 * i + 1
        drain(b0, 0)
        issue(b1, 1)
        compute_blk(b0, 0)
        drain(b1, 1)
        issue(b1 + 1, 0)
        compute_blk(b1, 1)
        return 0

    lax.fori_loop(0, NBLK // 2, pair, 0)

    drain(NBLK - 1, 0)
    compute_blk(NBLK - 1, 0)

    pltpu.sync_copy(obuf, out_hbm.at[pl.ds(wbase, PER_W)])


@jax.jit
def _run(src, dst, typ, z, rel_emb):
    mesh = plsc.VectorSubcoreMesh(core_axis_name="c", subcore_axis_name="s",
                                  num_cores=NC, num_subcores=NS)
    kern = pl.kernel(
        _body,
        out_type=jax.ShapeDtypeStruct((NUM_EDGES,), jnp.float32),
        mesh=mesh,
        compiler_params=pltpu.CompilerParams(needs_layout_passes=False,
                                             use_tc_tiling_on_sc=False),
        scratch_types=[
            pltpu.VMEM((PER_W,), jnp.int32),
            pltpu.VMEM((PER_W,), jnp.int32),
            pltpu.VMEM((PER_W,), jnp.int32),
            pltpu.VMEM((BLK, HW), jnp.int32),
            pltpu.VMEM((BLK, HW), jnp.int32),
            pltpu.VMEM((BLK, HW), jnp.int32),
            pltpu.VMEM((BLK, HW), jnp.int32),
            pltpu.VMEM((BLK, HW), jnp.int32),
            pltpu.VMEM((BLK, HW), jnp.int32),
            pltpu.VMEM((PER_W,), jnp.float32),
            pltpu.SemaphoreType.DMA,
            pltpu.SemaphoreType.DMA,
        ],
    )
    return kern(src, dst, typ, z, rel_emb)


def _as_i32_rows(t):
    n = t.shape[0]
    return lax.bitcast_convert_type(
        t.astype(jnp.bfloat16).reshape(n, HW, 2), jnp.int32)


def kernel(z, edge_index, edge_type, rel_emb):
    src = edge_index[0].astype(jnp.int32)
    dst = edge_index[1].astype(jnp.int32)
    typ = edge_type.astype(jnp.int32)
    return _run(src, dst, typ, _as_i32_rows(z), _as_i32_rows(rel_emb))


# speedup vs baseline: 4.4694x; 4.4694x over previous
import jax
import jax.numpy as jnp
from jax import lax
from jax.experimental import pallas as pl
from jax.experimental.pallas import tpu as pltpu
from jax.experimental.pallas import tpu_sc as plsc

NUM_EDGES = 320000
NUM_RELS = 500
HIDDEN = 128
HW = HIDDEN // 2
NCH = HW // 16
NC = 2
NS = 16
NW = NC * NS
PER_W = NUM_EDGES // NW
BLK = 80
NBLK = PER_W // BLK


def _body(src_hbm, dst_hbm, typ_hbm, z_hbm, rel_hbm, out_hbm,
          sidx, didx, tidx,
          srows0, drows0, rrows0, srows1, drows1, rrows1,
          obuf, sem0, sem1):
    wid = lax.axis_index("s") * NC + lax.axis_index("c")
    wbase = wid * PER_W

    bufs = ((srows0, drows0, rrows0, sem0), (srows1, drows1, rrows1, sem1))
    lanes = lax.iota(jnp.int32, 16)
    last_lane = lanes == 15

    pltpu.sync_copy(src_hbm.at[pl.ds(wbase, PER_W)], sidx)
    pltpu.sync_copy(dst_hbm.at[pl.ds(wbase, PER_W)], didx)
    pltpu.sync_copy(typ_hbm.at[pl.ds(wbase, PER_W)], tidx)

    def issue(b, parity):
        sb, db, rb, sem = bufs[parity]
        sl = pl.ds(b * BLK, BLK)
        pltpu.async_copy(z_hbm.at[sidx.at[sl]], sb, sem)
        pltpu.async_copy(z_hbm.at[didx.at[sl]], db, sem)
        pltpu.async_copy(rel_hbm.at[tidx.at[sl]], rb, sem)

    def drain(b, parity):
        sb, db, rb, sem = bufs[parity]
        sl = pl.ds(b * BLK, BLK)
        pltpu.make_async_copy(z_hbm.at[sidx.at[sl]], sb, sem).wait()
        pltpu.make_async_copy(z_hbm.at[didx.at[sl]], db, sem).wait()
        pltpu.make_async_copy(rel_hbm.at[tidx.at[sl]], rb, sem).wait()

    def compute_blk(b, parity):
        sb, db, rb, _ = bufs[parity]
        obase = b * BLK

        def group(g, _):
            rows = g * 16 + lanes

            def wstep(w, carry):
                cols, acc0, acc1 = carry
                s = plsc.bitcast(plsc.load_gather(sb, [rows, cols]),
                                 jnp.bfloat16)
                d = plsc.bitcast(plsc.load_gather(db, [rows, cols]),
                                 jnp.bfloat16)
                r = plsc.bitcast(plsc.load_gather(rb, [rows, cols]),
                                 jnp.bfloat16)
                p = s * d * r
                p0, p1 = plsc.unpack(p, format=plsc.PackFormat.INTERLEAVED)
                return (cols + 1) & (HW - 1), acc0 + p0, acc1 + p1

            z16 = jnp.zeros((16,), jnp.float32)
            _, acc0, acc1 = lax.fori_loop(0, HW, wstep, (lanes, z16, z16),
                                          unroll=2)
            obuf[pl.ds(obase + g * 16, 16)] = acc0 + acc1
            return 0

        lax.fori_loop(0, BLK // 16, group, 0)

    issue(0, 0)

    def pair(i, _):
        b0 = 2 * i
        b1 = 2 * i + 1
        drain(b0, 0)
        issue(b1, 1)
        compute_blk(b0, 0)
        drain(b1, 1)
        issue(b1 + 1, 0)
        compute_blk(b1, 1)
        return 0

    lax.fori_loop(0, NBLK // 2, pair, 0)

    drain(NBLK - 1, 0)
    compute_blk(NBLK - 1, 0)

    pltpu.sync_copy(obuf, out_hbm.at[pl.ds(wbase, PER_W)])


@jax.jit
def _run(src, dst, typ, z, rel_emb):
    mesh = plsc.VectorSubcoreMesh(core_axis_name="c", subcore_axis_name="s",
                                  num_cores=NC, num_subcores=NS)
    kern = pl.kernel(
        _body,
        out_type=jax.ShapeDtypeStruct((NUM_EDGES,), jnp.float32),
        mesh=mesh,
        compiler_params=pltpu.CompilerParams(needs_layout_passes=False,
                                             use_tc_tiling_on_sc=False),
        scratch_types=[
            pltpu.VMEM((PER_W,), jnp.int32),
            pltpu.VMEM((PER_W,), jnp.int32),
            pltpu.VMEM((PER_W,), jnp.int32),
            pltpu.VMEM((BLK, HW), jnp.int32),
            pltpu.VMEM((BLK, HW), jnp.int32),
            pltpu.VMEM((BLK, HW), jnp.int32),
            pltpu.VMEM((BLK, HW), jnp.int32),
            pltpu.VMEM((BLK, HW), jnp.int32),
            pltpu.VMEM((BLK, HW), jnp.int32),
            pltpu.VMEM((PER_W,), jnp.float32),
            pltpu.SemaphoreType.DMA,
            pltpu.SemaphoreType.DMA,
        ],
    )
    return kern(src, dst, typ, z, rel_emb)


def _as_i32_rows(t):
    n = t.shape[0]
    return lax.bitcast_convert_type(
        t.astype(jnp.bfloat16).reshape(n, HW, 2), jnp.int32)


def kernel(z, edge_index, edge_type, rel_emb):
    src = edge_index[0].astype(jnp.int32)
    dst = edge_index[1].astype(jnp.int32)
    typ = edge_type.astype(jnp.int32)
    return _run(src, dst, typ, _as_i32_rows(z), _as_i32_rows(rel_emb))


# speedup vs baseline: 5.1098x; 1.1433x over previous
import jax
import jax.numpy as jnp
from jax import lax
from jax.experimental import pallas as pl
from jax.experimental.pallas import tpu as pltpu
from jax.experimental.pallas import tpu_sc as plsc

NUM_EDGES = 320000
NUM_RELS = 500
HIDDEN = 128
HW = HIDDEN // 2
NCH = HW // 16
NC = 2
NS = 16
NW = NC * NS
PER_W = NUM_EDGES // NW
BLK = 80
NBLK = PER_W // BLK


def _body(src_hbm, dst_hbm, typ_hbm, z_hbm, rel_hbm, out_hbm,
          sidx, didx, tidx, rel_v,
          srows0, drows0, srows1, drows1,
          obuf, sem0, sem1):
    wid = lax.axis_index("s") * NC + lax.axis_index("c")
    wbase = wid * PER_W

    bufs = ((srows0, drows0, sem0), (srows1, drows1, sem1))
    lanes = lax.iota(jnp.int32, 16)
    last_lane = lanes == 15

    pltpu.sync_copy(src_hbm.at[pl.ds(wbase, PER_W)], sidx)
    pltpu.sync_copy(dst_hbm.at[pl.ds(wbase, PER_W)], didx)
    pltpu.sync_copy(typ_hbm.at[pl.ds(wbase, PER_W)], tidx)
    pltpu.sync_copy(rel_hbm, rel_v)

    def issue(b, parity):
        sb, db, sem = bufs[parity]
        sl = pl.ds(b * BLK, BLK)
        pltpu.async_copy(z_hbm.at[sidx.at[sl]], sb, sem)
        pltpu.async_copy(z_hbm.at[didx.at[sl]], db, sem)

    def drain(b, parity):
        sb, db, sem = bufs[parity]
        sl = pl.ds(b * BLK, BLK)
        pltpu.make_async_copy(z_hbm.at[sidx.at[sl]], sb, sem).wait()
        pltpu.make_async_copy(z_hbm.at[didx.at[sl]], db, sem).wait()

    def compute_blk(b, parity):
        sb, db, _ = bufs[parity]
        obase = b * BLK

        def group(g, _):
            rows = g * 16 + lanes
            tv = tidx[pl.ds(obase + g * 16, 16)]

            def wstep(w, carry):
                cols, acc0, acc1 = carry
                s = plsc.bitcast(plsc.load_gather(sb, [rows, cols]),
                                 jnp.bfloat16)
                d = plsc.bitcast(plsc.load_gather(db, [rows, cols]),
                                 jnp.bfloat16)
                r = plsc.bitcast(plsc.load_gather(rel_v, [tv, cols]),
                                 jnp.bfloat16)
                p = s * d * r
                p0, p1 = plsc.unpack(p, format=plsc.PackFormat.INTERLEAVED)
                return (cols + 1) & (HW - 1), acc0 + p0, acc1 + p1

            z16 = jnp.zeros((16,), jnp.float32)
            _, acc0, acc1 = lax.fori_loop(0, HW, wstep, (lanes, z16, z16),
                                          unroll=2)
            obuf[pl.ds(obase + g * 16, 16)] = acc0 + acc1
            return 0

        lax.fori_loop(0, BLK // 16, group, 0)

    issue(0, 0)

    def pair(i, _):
        b0 = 2 * i
        b1 = 2 * i + 1
        drain(b0, 0)
        issue(b1, 1)
        compute_blk(b0, 0)
        drain(b1, 1)
        issue(b1 + 1, 0)
        compute_blk(b1, 1)
        return 0

    lax.fori_loop(0, NBLK // 2, pair, 0)

    drain(NBLK - 1, 0)
    compute_blk(NBLK - 1, 0)

    pltpu.sync_copy(obuf, out_hbm.at[pl.ds(wbase, PER_W)])


@jax.jit
def _run(src, dst, typ, z, rel_emb):
    mesh = plsc.VectorSubcoreMesh(core_axis_name="c", subcore_axis_name="s",
                                  num_cores=NC, num_subcores=NS)
    kern = pl.kernel(
        _body,
        out_type=jax.ShapeDtypeStruct((NUM_EDGES,), jnp.float32),
        mesh=mesh,
        compiler_params=pltpu.CompilerParams(needs_layout_passes=False,
                                             use_tc_tiling_on_sc=False),
        scratch_types=[
            pltpu.VMEM((PER_W,), jnp.int32),
            pltpu.VMEM((PER_W,), jnp.int32),
            pltpu.VMEM((PER_W,), jnp.int32),
            pltpu.VMEM((NUM_RELS, HW), jnp.int32),
            pltpu.VMEM((BLK, HW), jnp.int32),
            pltpu.VMEM((BLK, HW), jnp.int32),
            pltpu.VMEM((BLK, HW), jnp.int32),
            pltpu.VMEM((BLK, HW), jnp.int32),
            pltpu.VMEM((PER_W,), jnp.float32),
            pltpu.SemaphoreType.DMA,
            pltpu.SemaphoreType.DMA,
        ],
    )
    return kern(src, dst, typ, z, rel_emb)


def _as_i32_rows(t):
    n = t.shape[0]
    return lax.bitcast_convert_type(
        t.astype(jnp.bfloat16).reshape(n, HW, 2), jnp.int32)


def kernel(z, edge_index, edge_type, rel_emb):
    src = edge_index[0].astype(jnp.int32)
    dst = edge_index[1].astype(jnp.int32)
    typ = edge_type.astype(jnp.int32)
    return _run(src, dst, typ, _as_i32_rows(z), _as_i32_rows(rel_emb))


# speedup vs baseline: 5.1102x; 1.0001x over previous
import jax
import jax.numpy as jnp
from jax import lax
from jax.experimental import pallas as pl
from jax.experimental.pallas import tpu as pltpu
from jax.experimental.pallas import tpu_sc as plsc

NUM_EDGES = 320000
NUM_RELS = 500
HIDDEN = 128
HW = HIDDEN // 2
NCH = HW // 16
NC = 2
NS = 16
NW = NC * NS
PER_W = NUM_EDGES // NW
BLK = 80
NBLK = PER_W // BLK


def _body(src_hbm, dst_hbm, typ_hbm, z_hbm, rel_hbm, out_hbm,
          sidx, didx, tidx, rel_v,
          srows0, drows0, srows1, drows1,
          obuf, sem0, sem1):
    wid = lax.axis_index("s") * NC + lax.axis_index("c")
    wbase = wid * PER_W

    bufs = ((srows0, drows0, sem0), (srows1, drows1, sem1))
    lanes = lax.iota(jnp.int32, 16)
    last_lane = lanes == 15

    pltpu.sync_copy(src_hbm.at[pl.ds(wbase, PER_W)], sidx)
    pltpu.sync_copy(dst_hbm.at[pl.ds(wbase, PER_W)], didx)
    pltpu.sync_copy(typ_hbm.at[pl.ds(wbase, PER_W)], tidx)
    pltpu.sync_copy(rel_hbm, rel_v)

    def issue(b, parity):
        sb, db, sem = bufs[parity]
        sl = pl.ds(b * BLK, BLK)
        pltpu.async_copy(z_hbm.at[sidx.at[sl]], sb, sem)
        pltpu.async_copy(z_hbm.at[didx.at[sl]], db, sem)

    def drain(b, parity):
        sb, db, sem = bufs[parity]
        sl = pl.ds(b * BLK, BLK)
        pltpu.make_async_copy(z_hbm.at[sidx.at[sl]], sb, sem).wait()
        pltpu.make_async_copy(z_hbm.at[didx.at[sl]], db, sem).wait()

    def compute_blk(b, parity):
        sb, db, _ = bufs[parity]
        obase = b * BLK

        def group(g, _):
            rows = g * 16 + lanes
            tv = tidx[pl.ds(obase + g * 16, 16)]

            def wstep(w, carry):
                cols, acc0, acc1 = carry
                s = plsc.bitcast(plsc.load_gather(sb, [rows, cols]),
                                 jnp.bfloat16)
                d = plsc.bitcast(plsc.load_gather(db, [rows, cols]),
                                 jnp.bfloat16)
                r = plsc.bitcast(plsc.load_gather(rel_v, [tv, cols]),
                                 jnp.bfloat16)
                p = s * d * r
                p0, p1 = plsc.unpack(p, format=plsc.PackFormat.INTERLEAVED)
                return (cols + 1) & (HW - 1), acc0 + p0, acc1 + p1

            z16 = jnp.zeros((16,), jnp.float32)
            _, acc0, acc1 = lax.fori_loop(0, HW, wstep, (lanes, z16, z16),
                                          unroll=4)
            obuf[pl.ds(obase + g * 16, 16)] = acc0 + acc1
            return 0

        lax.fori_loop(0, BLK // 16, group, 0)

    issue(0, 0)

    def pair(i, _):
        b0 = 2 * i
        b1 = 2 * i + 1
        drain(b0, 0)
        issue(b1, 1)
        compute_blk(b0, 0)
        drain(b1, 1)
        issue(b1 + 1, 0)
        compute_blk(b1, 1)
        return 0

    lax.fori_loop(0, NBLK // 2, pair, 0)

    drain(NBLK - 1, 0)
    compute_blk(NBLK - 1, 0)

    pltpu.sync_copy(obuf, out_hbm.at[pl.ds(wbase, PER_W)])


@jax.jit
def _run(src, dst, typ, z, rel_emb):
    mesh = plsc.VectorSubcoreMesh(core_axis_name="c", subcore_axis_name="s",
                                  num_cores=NC, num_subcores=NS)
    kern = pl.kernel(
        _body,
        out_type=jax.ShapeDtypeStruct((NUM_EDGES,), jnp.float32),
        mesh=mesh,
        compiler_params=pltpu.CompilerParams(needs_layout_passes=False,
                                             use_tc_tiling_on_sc=False),
        scratch_types=[
            pltpu.VMEM((PER_W,), jnp.int32),
            pltpu.VMEM((PER_W,), jnp.int32),
            pltpu.VMEM((PER_W,), jnp.int32),
            pltpu.VMEM((NUM_RELS, HW), jnp.int32),
            pltpu.VMEM((BLK, HW), jnp.int32),
            pltpu.VMEM((BLK, HW), jnp.int32),
            pltpu.VMEM((BLK, HW), jnp.int32),
            pltpu.VMEM((BLK, HW), jnp.int32),
            pltpu.VMEM((PER_W,), jnp.float32),
            pltpu.SemaphoreType.DMA,
            pltpu.SemaphoreType.DMA,
        ],
    )
    return kern(src, dst, typ, z, rel_emb)


def _as_i32_rows(t):
    n = t.shape[0]
    return lax.bitcast_convert_type(
        t.astype(jnp.bfloat16).reshape(n, HW, 2), jnp.int32)


def kernel(z, edge_index, edge_type, rel_emb):
    src = edge_index[0].astype(jnp.int32)
    dst = edge_index[1].astype(jnp.int32)
    typ = edge_type.astype(jnp.int32)
    return _run(src, dst, typ, _as_i32_rows(z), _as_i32_rows(rel_emb))
